# baseline (speedup 1.0000x reference)
import math

import jax
import jax.numpy as jnp
from jax import lax
from jax.experimental import pallas as pl
from jax.experimental.pallas import tpu as pltpu
from jax.experimental.pallas import tpu_sc as plsc

N = 10000
E = 320000
F = 128

NC = 2
NS = 16
NW = NC * NS
EPW = E // NW
CH = 80
NIT = EPW // CH
GRP = CH // 16


FW = F // 2


def _pack(a):
    lo = jax.lax.bitcast_convert_type(
        a[:, :FW].astype(jnp.bfloat16), jnp.uint16).astype(jnp.uint32)
    hi = jax.lax.bitcast_convert_type(
        a[:, FW:].astype(jnp.bfloat16), jnp.uint16).astype(jnp.uint32)
    return (lo | (hi << 16)).astype(jnp.int32)


def _mm_body(x_ref, wq_ref, wk_ref, q_ref, k_ref):
    xb = x_ref[...]
    scale = 1.0 / math.sqrt(F)
    q = jnp.dot(xb, wq_ref[...], preferred_element_type=jnp.float32) * scale
    k = jnp.dot(xb, wk_ref[...], preferred_element_type=jnp.float32)
    q_ref[...] = _pack(q)
    k_ref[...] = _pack(k)


def _project(x, Wq, Wk):
    blk = 1000
    return pl.pallas_call(
        _mm_body,
        grid=(N // blk,),
        in_specs=[
            pl.BlockSpec((blk, F), lambda i: (i, 0)),
            pl.BlockSpec((F, F), lambda i: (0, 0)),
            pl.BlockSpec((F, F), lambda i: (0, 0)),
        ],
        out_specs=[
            pl.BlockSpec((blk, FW), lambda i: (i, 0)),
            pl.BlockSpec((blk, FW), lambda i: (i, 0)),
        ],
        out_shape=[
            jax.ShapeDtypeStruct((N, FW), jnp.int32),
            jax.ShapeDtypeStruct((N, FW), jnp.int32),
        ],
    )(x, Wq, Wk)


DEPTH = 2


def _sc_body(q_hbm, k_hbm, ii_hbm, jj_hbm, out_hbm,
             k_sh,
             ii_v, jj_v, qr0, kr0, qr1, kr1, out_v,
             sq0, sk0, sq1, sk1):
    c = lax.axis_index("c")
    s = lax.axis_index("s")
    wid = s * NC + c

    nrows = N // NS
    pltpu.sync_copy(k_hbm.at[pl.ds(s * nrows, nrows)],
                    k_sh.at[pl.ds(s * nrows, nrows)])

    pltpu.sync_copy(ii_hbm.at[wid], ii_v)
    pltpu.sync_copy(jj_hbm.at[wid], jj_v)
    plsc.subcore_barrier()

    bufs = ((qr0, kr0, sq0, sk0), (qr1, kr1, sq1, sk1))

    def start(it, b):
        qr, kr, sq, sk = bufs[b]
        pltpu.async_copy(q_hbm.at[ii_v.at[it]], qr, sq)
        pltpu.async_copy(k_sh.at[jj_v.at[it]], kr, sk)

    def wait(it, b):
        qr, kr, sq, sk = bufs[b]
        pltpu.make_async_copy(q_hbm.at[ii_v.at[it]], qr, sq).wait()
        pltpu.make_async_copy(k_sh.at[jj_v.at[it]], kr, sk).wait()

    def compute(it, b):
        qr, kr, _, _ = bufs[b]

        def grp_body(g, carry2):
            lane = lax.iota(jnp.int32, 16)
            rows = g * 16 + lane

            def feat(w, accs):
                acc_lo, acc_hi = accs
                colw = (w + lane) & (FW - 1)
                qw = plsc.load_gather(qr, [rows, colw])
                kw = plsc.load_gather(kr, [rows, colw])
                q_lo = plsc.bitcast(qw << 16, jnp.float32)
                k_lo = plsc.bitcast(kw << 16, jnp.float32)
                q_hi = plsc.bitcast(qw & -65536, jnp.float32)
                k_hi = plsc.bitcast(kw & -65536, jnp.float32)
                return acc_lo + q_lo * k_lo, acc_hi + q_hi * k_hi

            zero = jnp.zeros((16,), jnp.float32)
            acc_lo, acc_hi = lax.fori_loop(0, FW, feat, (zero, zero),
                                           unroll=16)
            out_v[pl.ds(it * CH + g * 16, 16)] = acc_lo + acc_hi
            return carry2

        lax.fori_loop(0, GRP, grp_body, 0)

    for it in range(DEPTH - 1):
        start(it, it % DEPTH)

    def ring_body(p, carry):
        for b in range(DEPTH):
            it = p * DEPTH + b

            @pl.when(it + DEPTH - 1 < NIT)
            def _():
                start(it + DEPTH - 1, (b + DEPTH - 1) % DEPTH)

            wait(it, b)
            compute(it, b)
        return carry

    lax.fori_loop(0, NIT // DEPTH, ring_body, 0)
    for it in range((NIT // DEPTH) * DEPTH, NIT):
        wait(it, it % DEPTH)
        compute(it, it % DEPTH)

    pltpu.sync_copy(out_v, out_hbm.at[pl.ds(wid * EPW, EPW)])


def _edge_scores(q, k, ii, jj):
    mesh = plsc.VectorSubcoreMesh(core_axis_name="c", subcore_axis_name="s")
    fn = pl.kernel(
        _sc_body,
        out_type=jax.ShapeDtypeStruct((E,), jnp.float32),
        mesh=mesh,
        scratch_types=[
            pltpu.VMEM_SHARED((N, FW), jnp.int32),
            pltpu.VMEM((NIT, CH), jnp.int32),
            pltpu.VMEM((NIT, CH), jnp.int32),
            pltpu.VMEM((CH, FW), jnp.int32),
            pltpu.VMEM((CH, FW), jnp.int32),
            pltpu.VMEM((CH, FW), jnp.int32),
            pltpu.VMEM((CH, FW), jnp.int32),
            pltpu.VMEM((EPW,), jnp.float32),
            pltpu.SemaphoreType.DMA,
            pltpu.SemaphoreType.DMA,
            pltpu.SemaphoreType.DMA,
            pltpu.SemaphoreType.DMA,
        ],
        compiler_params=pltpu.CompilerParams(needs_layout_passes=False,
                                             use_tc_tiling_on_sc=False),
    )
    return fn(q, k, ii, jj)


def kernel(x, idx_i, idx_j, Wq, Wk):
    q, k = _project(x, Wq, Wk)
    ii = idx_i.reshape(NW, NIT, CH)
    jj = idx_j.reshape(NW, NIT, CH)
    return _edge_scores(q, k, ii, jj)


# speedup vs baseline: 1.3034x; 1.3034x over previous
import math

import jax
import jax.numpy as jnp
from jax import lax
from jax.experimental import pallas as pl
from jax.experimental.pallas import tpu as pltpu
from jax.experimental.pallas import tpu_sc as plsc

N = 10000
E = 320000
F = 128

NC = 2
NS = 16
NW = NC * NS
EPW = E // NW
CH = 80
NIT = EPW // CH
GRP = CH // 16


FW = F // 2


def _pack(a):
    lo = jax.lax.bitcast_convert_type(
        a[:, :FW].astype(jnp.bfloat16), jnp.uint16).astype(jnp.uint32)
    hi = jax.lax.bitcast_convert_type(
        a[:, FW:].astype(jnp.bfloat16), jnp.uint16).astype(jnp.uint32)
    return (lo | (hi << 16)).astype(jnp.int32)


def _mm_body(x_ref, wq_ref, wk_ref, q_ref, k_ref):
    xb = x_ref[...]
    scale = 1.0 / math.sqrt(F)
    q = jnp.dot(xb, wq_ref[...], preferred_element_type=jnp.float32) * scale
    k = jnp.dot(xb, wk_ref[...], preferred_element_type=jnp.float32)
    q_ref[...] = _pack(q)
    k_ref[...] = _pack(k)


def _project(x, Wq, Wk):
    blk = 1000
    return pl.pallas_call(
        _mm_body,
        grid=(N // blk,),
        in_specs=[
            pl.BlockSpec((blk, F), lambda i: (i, 0)),
            pl.BlockSpec((F, F), lambda i: (0, 0)),
            pl.BlockSpec((F, F), lambda i: (0, 0)),
        ],
        out_specs=[
            pl.BlockSpec((blk, FW), lambda i: (i, 0)),
            pl.BlockSpec((blk, FW), lambda i: (i, 0)),
        ],
        out_shape=[
            jax.ShapeDtypeStruct((N, FW), jnp.int32),
            jax.ShapeDtypeStruct((N, FW), jnp.int32),
        ],
    )(x, Wq, Wk)


DEPTH = 6


def _sc_body(q_hbm, k_hbm, ii_hbm, jj_hbm, out_hbm,
             ii_v, jj_v, qr0, kr0, qr1, kr1, qr2, kr2, qr3, kr3,
             qr4, kr4, qr5, kr5, out_v,
             sq0, sk0, sq1, sk1, sq2, sk2, sq3, sk3, sq4, sk4, sq5, sk5):
    c = lax.axis_index("c")
    s = lax.axis_index("s")
    wid = s * NC + c

    pltpu.sync_copy(ii_hbm.at[wid], ii_v)
    pltpu.sync_copy(jj_hbm.at[wid], jj_v)

    bufs = ((qr0, kr0, sq0, sk0), (qr1, kr1, sq1, sk1),
            (qr2, kr2, sq2, sk2), (qr3, kr3, sq3, sk3),
            (qr4, kr4, sq4, sk4), (qr5, kr5, sq5, sk5))

    def start(it, b):
        qr, kr, sq, sk = bufs[b]
        pltpu.async_copy(q_hbm.at[ii_v.at[it]], qr, sq)
        pltpu.async_copy(k_hbm.at[jj_v.at[it]], kr, sk)

    def wait(it, b):
        qr, kr, sq, sk = bufs[b]
        pltpu.make_async_copy(q_hbm.at[ii_v.at[it]], qr, sq).wait()
        pltpu.make_async_copy(k_hbm.at[jj_v.at[it]], kr, sk).wait()

    def compute(it, b):
        qr, kr, _, _ = bufs[b]

        def grp_body(g, carry2):
            lane = lax.iota(jnp.int32, 16)
            rows = g * 16 + lane

            def feat(w, accs):
                acc_lo, acc_hi = accs
                colw = (w + lane) & (FW - 1)
                qw = plsc.load_gather(qr, [rows, colw])
                kw = plsc.load_gather(kr, [rows, colw])
                pw = plsc.bitcast(plsc.bitcast(qw, jnp.bfloat16) *
                                  plsc.bitcast(kw, jnp.bfloat16), jnp.int32)
                p_lo = plsc.bitcast(pw << 16, jnp.float32)
                p_hi = plsc.bitcast(pw & -65536, jnp.float32)
                return acc_lo + p_lo, acc_hi + p_hi

            zero = jnp.zeros((16,), jnp.float32)
            acc_lo, acc_hi = lax.fori_loop(0, FW, feat, (zero, zero),
                                           unroll=16)
            out_v[pl.ds(it * CH + g * 16, 16)] = acc_lo + acc_hi
            return carry2

        lax.fori_loop(0, GRP, grp_body, 0)

    for it in range(DEPTH - 1):
        start(it, it % DEPTH)

    def ring_body(p, carry):
        for b in range(DEPTH):
            it = p * DEPTH + b

            @pl.when(it + DEPTH - 1 < NIT)
            def _():
                start(it + DEPTH - 1, (b + DEPTH - 1) % DEPTH)

            wait(it, b)
            compute(it, b)
        return carry

    lax.fori_loop(0, NIT // DEPTH, ring_body, 0)
    for it in range((NIT // DEPTH) * DEPTH, NIT):
        wait(it, it % DEPTH)
        compute(it, it % DEPTH)

    pltpu.sync_copy(out_v, out_hbm.at[pl.ds(wid * EPW, EPW)])


def _edge_scores(q, k, ii, jj):
    mesh = plsc.VectorSubcoreMesh(core_axis_name="c", subcore_axis_name="s")
    fn = pl.kernel(
        _sc_body,
        out_type=jax.ShapeDtypeStruct((E,), jnp.float32),
        mesh=mesh,
        scratch_types=[
            pltpu.VMEM((NIT, CH), jnp.int32),
            pltpu.VMEM((NIT, CH), jnp.int32),
            pltpu.VMEM((CH, FW), jnp.int32),
            pltpu.VMEM((CH, FW), jnp.int32),
            pltpu.VMEM((CH, FW), jnp.int32),
            pltpu.VMEM((CH, FW), jnp.int32),
            pltpu.VMEM((CH, FW), jnp.int32),
            pltpu.VMEM((CH, FW), jnp.int32),
            pltpu.VMEM((CH, FW), jnp.int32),
            pltpu.VMEM((CH, FW), jnp.int32),
            pltpu.VMEM((CH, FW), jnp.int32),
            pltpu.VMEM((CH, FW), jnp.int32),
            pltpu.VMEM((CH, FW), jnp.int32),
            pltpu.VMEM((CH, FW), jnp.int32),
            pltpu.VMEM((EPW,), jnp.float32),
        ] + [pltpu.SemaphoreType.DMA] * 12,
        compiler_params=pltpu.CompilerParams(needs_layout_passes=False,
                                             use_tc_tiling_on_sc=False),
    )
    return fn(q, k, ii, jj)


def kernel(x, idx_i, idx_j, Wq, Wk):
    q, k = _project(x, Wq, Wk)
    ii = idx_i.reshape(NW, NIT, CH)
    jj = idx_j.reshape(NW, NIT, CH)
    return _edge_scores(q, k, ii, jj)


# speedup vs baseline: 1.3602x; 1.0436x over previous
import math

import jax
import jax.numpy as jnp
from jax import lax
from jax.experimental import pallas as pl
from jax.experimental.pallas import tpu as pltpu
from jax.experimental.pallas import tpu_sc as plsc

N = 10000
E = 320000
F = 128

NC = 2
NS = 16
NW = NC * NS
EPW = E // NW
CH = 80
NIT = EPW // CH
GRP = CH // 16


FW = F // 2


def _pack(a):
    lo = jax.lax.bitcast_convert_type(
        a[:, :FW].astype(jnp.bfloat16), jnp.uint16).astype(jnp.uint32)
    hi = jax.lax.bitcast_convert_type(
        a[:, FW:].astype(jnp.bfloat16), jnp.uint16).astype(jnp.uint32)
    return (lo | (hi << 16)).astype(jnp.int32)


def _mm_body(x_ref, wq_ref, wk_ref, qk_ref):
    xb = x_ref[...]
    scale = 1.0 / math.sqrt(F)
    q = jnp.dot(xb, wq_ref[...], preferred_element_type=jnp.float32) * scale
    k = jnp.dot(xb, wk_ref[...], preferred_element_type=jnp.float32)
    qk_ref[...] = jnp.concatenate([_pack(q), _pack(k)], axis=1)


def _project(x, Wq, Wk):
    blk = 1000
    return pl.pallas_call(
        _mm_body,
        grid=(N // blk,),
        in_specs=[
            pl.BlockSpec((blk, F), lambda i: (i, 0)),
            pl.BlockSpec((F, F), lambda i: (0, 0)),
            pl.BlockSpec((F, F), lambda i: (0, 0)),
        ],
        out_specs=pl.BlockSpec((blk, 2 * FW), lambda i: (i, 0)),
        out_shape=jax.ShapeDtypeStruct((N, 2 * FW), jnp.int32),
    )(x, Wq, Wk)


DEPTH = 6


def _sc_body(qk_hbm, ii_hbm, jj_hbm, out_hbm,
             ii_v, jj_v, qr0, kr0, qr1, kr1, qr2, kr2, qr3, kr3,
             qr4, kr4, qr5, kr5, out_v,
             sq0, sk0, sq1, sk1, sq2, sk2, sq3, sk3, sq4, sk4, sq5, sk5):
    c = lax.axis_index("c")
    s = lax.axis_index("s")
    wid = s * NC + c

    pltpu.sync_copy(ii_hbm.at[wid], ii_v)
    pltpu.sync_copy(jj_hbm.at[wid], jj_v)

    def tbody(t, carry):
        r = t // GRP
        col = (t % GRP) * 16
        ii_v[r, pl.ds(col, 16)] = ii_v[r, pl.ds(col, 16)] * 2
        jj_v[r, pl.ds(col, 16)] = jj_v[r, pl.ds(col, 16)] * 2 + 1
        return carry

    lax.fori_loop(0, NIT * GRP, tbody, 0, unroll=8)

    bufs = ((qr0, kr0, sq0, sk0), (qr1, kr1, sq1, sk1),
            (qr2, kr2, sq2, sk2), (qr3, kr3, sq3, sk3),
            (qr4, kr4, sq4, sk4), (qr5, kr5, sq5, sk5))

    def start(it, b):
        qr, kr, sq, sk = bufs[b]
        pltpu.async_copy(qk_hbm.at[ii_v.at[it]], qr, sq)
        pltpu.async_copy(qk_hbm.at[jj_v.at[it]], kr, sk)

    def wait(it, b):
        qr, kr, sq, sk = bufs[b]
        pltpu.make_async_copy(qk_hbm.at[ii_v.at[it]], qr, sq).wait()
        pltpu.make_async_copy(qk_hbm.at[jj_v.at[it]], kr, sk).wait()

    def compute(it, b):
        qr, kr, _, _ = bufs[b]

        def grp_body(g, carry2):
            lane = lax.iota(jnp.int32, 16)
            rows = g * 16 + lane

            def feat(w, accs):
                acc_lo, acc_hi = accs
                colw = (w + lane) & (FW - 1)
                qw = plsc.load_gather(qr, [rows, colw])
                kw = plsc.load_gather(kr, [rows, colw])
                pw = plsc.bitcast(plsc.bitcast(qw, jnp.bfloat16) *
                                  plsc.bitcast(kw, jnp.bfloat16), jnp.int32)
                p_lo = plsc.bitcast(pw << 16, jnp.float32)
                p_hi = plsc.bitcast(pw & -65536, jnp.float32)
                return acc_lo + p_lo, acc_hi + p_hi

            zero = jnp.zeros((16,), jnp.float32)
            acc_lo, acc_hi = lax.fori_loop(0, FW, feat, (zero, zero),
                                           unroll=16)
            out_v[pl.ds(it * CH + g * 16, 16)] = acc_lo + acc_hi
            return carry2

        lax.fori_loop(0, GRP, grp_body, 0)

    for it in range(DEPTH - 1):
        start(it, it % DEPTH)

    def ring_body(p, carry):
        for b in range(DEPTH):
            it = p * DEPTH + b

            @pl.when(it + DEPTH - 1 < NIT)
            def _():
                start(it + DEPTH - 1, (b + DEPTH - 1) % DEPTH)

            wait(it, b)
            compute(it, b)
        return carry

    lax.fori_loop(0, NIT // DEPTH, ring_body, 0)
    for it in range((NIT // DEPTH) * DEPTH, NIT):
        wait(it, it % DEPTH)
        compute(it, it % DEPTH)

    pltpu.sync_copy(out_v, out_hbm.at[pl.ds(wid * EPW, EPW)])


def _edge_scores(qk, ii, jj):
    mesh = plsc.VectorSubcoreMesh(core_axis_name="c", subcore_axis_name="s")
    fn = pl.kernel(
        _sc_body,
        out_type=jax.ShapeDtypeStruct((E,), jnp.float32),
        mesh=mesh,
        scratch_types=[
            pltpu.VMEM((NIT, CH), jnp.int32),
            pltpu.VMEM((NIT, CH), jnp.int32),
            pltpu.VMEM((CH, FW), jnp.int32),
            pltpu.VMEM((CH, FW), jnp.int32),
            pltpu.VMEM((CH, FW), jnp.int32),
            pltpu.VMEM((CH, FW), jnp.int32),
            pltpu.VMEM((CH, FW), jnp.int32),
            pltpu.VMEM((CH, FW), jnp.int32),
            pltpu.VMEM((CH, FW), jnp.int32),
            pltpu.VMEM((CH, FW), jnp.int32),
            pltpu.VMEM((CH, FW), jnp.int32),
            pltpu.VMEM((CH, FW), jnp.int32),
            pltpu.VMEM((CH, FW), jnp.int32),
            pltpu.VMEM((CH, FW), jnp.int32),
            pltpu.VMEM((EPW,), jnp.float32),
        ] + [pltpu.SemaphoreType.DMA] * 12,
        compiler_params=pltpu.CompilerParams(needs_layout_passes=False,
                                             use_tc_tiling_on_sc=False),
    )
    return fn(qk, ii, jj)


def kernel(x, idx_i, idx_j, Wq, Wk):
    qk = _project(x, Wq, Wk)
    qk2 = qk.reshape(2 * N, FW)
    ii = idx_i.reshape(NW, NIT, CH)
    jj = idx_j.reshape(NW, NIT, CH)
    return _edge_scores(qk2, ii, jj)
